# baseline (speedup 1.0000x reference)
import functools

import jax
import jax.numpy as jnp
from jax.experimental import pallas as pl
from jax.experimental.pallas import tpu as pltpu

NUM_CODES = 1024
DIM = 64
PIX = 1024
COMMIT = 0.25


def _vq_kernel(x_ref, w_ref, q_ref, loss_ref):
    b = pl.program_id(0)
    x = x_ref[0]
    w = w_ref[...]

    xt = x.T
    wsq = jnp.sum(w * w, axis=1)[None, :]
    xsq = jnp.sum(xt * xt, axis=1, keepdims=True)
    xw = jax.lax.dot_general(
        xt, w, (((1,), (1,)), ((), ())),
        preferred_element_type=jnp.float32)
    d2 = (xsq - 2.0 * xw) + wsq
    scores = jnp.sqrt(jnp.maximum(d2, 0.0))

    m = jnp.min(scores, axis=1, keepdims=True)
    iota = jax.lax.broadcasted_iota(jnp.int32, scores.shape, 1)
    masked = jnp.where(scores <= m, iota, NUM_CODES)
    idx = jnp.min(masked, axis=1, keepdims=True)

    onehot = (iota == idx).astype(jnp.float32)
    q = jax.lax.dot_general(
        onehot, w, (((1,), (0,)), ((), ())),
        preferred_element_type=jnp.float32,
        precision=jax.lax.Precision.HIGHEST)
    q_ref[0] = q.T

    diff = q - xt
    partial = jnp.full((8, 128), jnp.sum(diff * diff), jnp.float32)

    @pl.when(b == 0)
    def _init():
        loss_ref[...] = jnp.zeros((8, 128), jnp.float32)

    loss_ref[...] += partial


@jax.jit
def kernel(inputs, weight):
    B, C, H, W = inputs.shape
    x = inputs.reshape(B, C, H * W)

    q, loss = pl.pallas_call(
        _vq_kernel,
        grid=(B,),
        in_specs=[
            pl.BlockSpec((1, C, H * W), lambda b: (b, 0, 0)),
            pl.BlockSpec((NUM_CODES, DIM), lambda b: (0, 0)),
        ],
        out_specs=[
            pl.BlockSpec((1, C, H * W), lambda b: (b, 0, 0)),
            pl.BlockSpec((8, 128), lambda b: (0, 0)),
        ],
        out_shape=[
            jax.ShapeDtypeStruct((B, C, H * W), jnp.float32),
            jax.ShapeDtypeStruct((8, 128), jnp.float32),
        ],
        compiler_params=pltpu.CompilerParams(
            dimension_semantics=("arbitrary",),
        ),
    )(x, weight)

    n = B * C * H * W
    c_loss = (1.0 + COMMIT) * loss[0, 0] / n
    return c_loss, q.reshape(B, C, H, W)


# speedup vs baseline: 1.5933x; 1.5933x over previous
import functools

import jax
import jax.numpy as jnp
from jax.experimental import pallas as pl
from jax.experimental.pallas import tpu as pltpu

NUM_CODES = 1024
DIM = 64
PIX = 1024
COMMIT = 0.25


def _vq_kernel(x_ref, w_ref, q_ref, loss_ref):
    b = pl.program_id(0)
    x = x_ref[0]
    w = w_ref[...]

    xt = x.T
    wsq = jnp.sum(w * w, axis=1)[None, :]
    xsq = jnp.sum(xt * xt, axis=1, keepdims=True)
    xw = jax.lax.dot_general(
        xt, w, (((1,), (1,)), ((), ())),
        preferred_element_type=jnp.float32)
    d2 = (xsq - 2.0 * xw) + wsq
    scores = jnp.sqrt(jnp.maximum(d2, 0.0))

    m = jnp.min(scores, axis=1, keepdims=True)
    iota = jax.lax.broadcasted_iota(jnp.int32, scores.shape, 1)
    masked = jnp.where(scores <= m, iota, NUM_CODES)
    idx = jnp.min(masked, axis=1, keepdims=True)

    onehot = (iota == idx).astype(jnp.float32)
    q = jax.lax.dot_general(
        onehot, w, (((1,), (0,)), ((), ())),
        preferred_element_type=jnp.float32)
    q_ref[0] = q.T

    diff = q - xt
    partial = jnp.full((8, 128), jnp.sum(diff * diff), jnp.float32)

    @pl.when(b == 0)
    def _init():
        loss_ref[...] = jnp.zeros((8, 128), jnp.float32)

    loss_ref[...] += partial


@jax.jit
def kernel(inputs, weight):
    B, C, H, W = inputs.shape
    x = inputs.reshape(B, C, H * W)

    q, loss = pl.pallas_call(
        _vq_kernel,
        grid=(B,),
        in_specs=[
            pl.BlockSpec((1, C, H * W), lambda b: (b, 0, 0)),
            pl.BlockSpec((NUM_CODES, DIM), lambda b: (0, 0)),
        ],
        out_specs=[
            pl.BlockSpec((1, C, H * W), lambda b: (b, 0, 0)),
            pl.BlockSpec((8, 128), lambda b: (0, 0)),
        ],
        out_shape=[
            jax.ShapeDtypeStruct((B, C, H * W), jnp.float32),
            jax.ShapeDtypeStruct((8, 128), jnp.float32),
        ],
        compiler_params=pltpu.CompilerParams(
            dimension_semantics=("arbitrary",),
        ),
    )(x, weight)

    n = B * C * H * W
    c_loss = (1.0 + COMMIT) * loss[0, 0] / n
    return c_loss, q.reshape(B, C, H, W)


# speedup vs baseline: 1.6730x; 1.0500x over previous
import functools

import jax
import jax.numpy as jnp
from jax.experimental import pallas as pl
from jax.experimental.pallas import tpu as pltpu

NUM_CODES = 1024
DIM = 64
PIX = 1024
COMMIT = 0.25


def _vq_kernel(x_ref, w_ref, q_ref, loss_ref):
    b = pl.program_id(0)
    x = x_ref[...]
    w = w_ref[...]
    bb, _, pix = x.shape
    rows = bb * pix

    xt = x.transpose(0, 2, 1).reshape(rows, DIM)
    wsq = jnp.sum(w * w, axis=1)[None, :]
    xsq = jnp.sum(xt * xt, axis=1, keepdims=True)
    xw = jax.lax.dot_general(
        xt, w, (((1,), (1,)), ((), ())),
        preferred_element_type=jnp.float32)
    d2 = (xsq - 2.0 * xw) + wsq
    scores = jnp.sqrt(jnp.maximum(d2, 0.0))

    m = jnp.min(scores, axis=1, keepdims=True)
    iota = jax.lax.broadcasted_iota(jnp.int32, scores.shape, 1)
    masked = jnp.where(scores <= m, iota, NUM_CODES)
    idx = jnp.min(masked, axis=1, keepdims=True)

    onehot = (iota == idx).astype(jnp.float32)
    q = jax.lax.dot_general(
        onehot, w, (((1,), (0,)), ((), ())),
        preferred_element_type=jnp.float32)
    q_ref[...] = q.reshape(bb, pix, DIM).transpose(0, 2, 1)

    diff = q - xt
    partial = jnp.full((8, 128), jnp.sum(diff * diff), jnp.float32)

    @pl.when(b == 0)
    def _init():
        loss_ref[...] = jnp.zeros((8, 128), jnp.float32)

    loss_ref[...] += partial


BATCH_BLOCK = 2


@jax.jit
def kernel(inputs, weight):
    B, C, H, W = inputs.shape
    x = inputs.reshape(B, C, H * W)

    q, loss = pl.pallas_call(
        _vq_kernel,
        grid=(B // BATCH_BLOCK,),
        in_specs=[
            pl.BlockSpec((BATCH_BLOCK, C, H * W), lambda b: (b, 0, 0)),
            pl.BlockSpec((NUM_CODES, DIM), lambda b: (0, 0)),
        ],
        out_specs=[
            pl.BlockSpec((BATCH_BLOCK, C, H * W), lambda b: (b, 0, 0)),
            pl.BlockSpec((8, 128), lambda b: (0, 0)),
        ],
        out_shape=[
            jax.ShapeDtypeStruct((B, C, H * W), jnp.float32),
            jax.ShapeDtypeStruct((8, 128), jnp.float32),
        ],
        compiler_params=pltpu.CompilerParams(
            dimension_semantics=("arbitrary",),
        ),
    )(x, weight)

    n = B * C * H * W
    c_loss = (1.0 + COMMIT) * loss[0, 0] / n
    return c_loss, q.reshape(B, C, H, W)


# speedup vs baseline: 1.9407x; 1.1600x over previous
import jax
import jax.numpy as jnp
from jax.experimental import pallas as pl
from jax.experimental.pallas import tpu as pltpu

NUM_CODES = 1024
DIM = 64
COMMIT = 0.25


def _vq_kernel(x_ref, w_ref, q_ref, msq_ref):
    x = x_ref[0]
    w = w_ref[...]
    wsq = jnp.sum(w * w, axis=1)[:, None]
    xsq = jnp.sum(x * x, axis=0, keepdims=True)
    xw = jax.lax.dot_general(
        w, x, (((1,), (0,)), ((), ())),
        preferred_element_type=jnp.float32)
    d2 = (xsq - 2.0 * xw) + wsq
    scores = jnp.sqrt(jnp.maximum(d2, 0.0))

    m = jnp.min(scores, axis=0, keepdims=True)
    iota = jax.lax.broadcasted_iota(jnp.int32, scores.shape, 0)
    masked = jnp.where(scores <= m, iota, NUM_CODES)
    idx = jnp.min(masked, axis=0, keepdims=True)

    onehot = (iota == idx).astype(jnp.float32)
    q = jax.lax.dot_general(
        w, onehot, (((0,), (0,)), ((), ())),
        preferred_element_type=jnp.float32)
    q_ref[0] = q
    msq_ref[0] = m * m


def _loss_kernel(msq_ref, loss_ref):
    loss_ref[...] = jnp.full((8, 128), jnp.sum(msq_ref[...]), jnp.float32)


@jax.jit
def kernel(inputs, weight):
    B, C, H, W = inputs.shape
    x = inputs.reshape(B, C, H * W)

    q, msq = pl.pallas_call(
        _vq_kernel,
        grid=(B,),
        in_specs=[
            pl.BlockSpec((1, C, H * W), lambda b: (b, 0, 0)),
            pl.BlockSpec((NUM_CODES, DIM), lambda b: (0, 0)),
        ],
        out_specs=[
            pl.BlockSpec((1, C, H * W), lambda b: (b, 0, 0)),
            pl.BlockSpec((1, 1, H * W), lambda b: (b, 0, 0)),
        ],
        out_shape=[
            jax.ShapeDtypeStruct((B, C, H * W), jnp.float32),
            jax.ShapeDtypeStruct((B, 1, H * W), jnp.float32),
        ],
        compiler_params=pltpu.CompilerParams(
            dimension_semantics=("arbitrary",),
        ),
    )(x, weight)

    loss = pl.pallas_call(
        _loss_kernel,
        out_shape=jax.ShapeDtypeStruct((8, 128), jnp.float32),
    )(msq)

    n = B * C * H * W
    c_loss = (1.0 + COMMIT) * loss[0, 0] / n
    return c_loss, q.reshape(B, C, H, W)


# speedup vs baseline: 2.1134x; 1.0890x over previous
import jax
import jax.numpy as jnp
from jax.experimental import pallas as pl
from jax.experimental.pallas import tpu as pltpu

NUM_CODES = 1024
DIM = 64
COMMIT = 0.25
ROW_BLOCK = 2048


def _vq_kernel(xt_ref, w_ref, q_ref, loss_ref):
    b = pl.program_id(0)
    xt = xt_ref[...]
    w = w_ref[...]

    wsq = jnp.sum(w * w, axis=1)[None, :]
    xsq = jnp.sum(xt * xt, axis=1, keepdims=True)
    xw = jax.lax.dot_general(
        xt, w, (((1,), (1,)), ((), ())),
        preferred_element_type=jnp.float32)
    d2 = (xsq - 2.0 * xw) + wsq
    scores = jnp.sqrt(jnp.maximum(d2, 0.0))

    m = jnp.min(scores, axis=1, keepdims=True)
    iota = jax.lax.broadcasted_iota(jnp.int32, scores.shape, 1)
    masked = jnp.where(scores <= m, iota, NUM_CODES)
    idx = jnp.min(masked, axis=1, keepdims=True)

    onehot = (iota == idx).astype(jnp.float32)
    q = jax.lax.dot_general(
        onehot, w, (((1,), (0,)), ((), ())),
        preferred_element_type=jnp.float32)
    q_ref[...] = q

    partial = jnp.full((8, 128), jnp.sum(m * m), jnp.float32)

    @pl.when(b == 0)
    def _init():
        loss_ref[...] = jnp.zeros((8, 128), jnp.float32)

    loss_ref[...] += partial


@jax.jit
def kernel(inputs, weight):
    B, C, H, W = inputs.shape
    rows = B * H * W
    xt = inputs.transpose(0, 2, 3, 1).reshape(rows, C)

    q, loss = pl.pallas_call(
        _vq_kernel,
        grid=(rows // ROW_BLOCK,),
        in_specs=[
            pl.BlockSpec((ROW_BLOCK, DIM), lambda b: (b, 0)),
            pl.BlockSpec((NUM_CODES, DIM), lambda b: (0, 0)),
        ],
        out_specs=[
            pl.BlockSpec((ROW_BLOCK, DIM), lambda b: (b, 0)),
            pl.BlockSpec((8, 128), lambda b: (0, 0)),
        ],
        out_shape=[
            jax.ShapeDtypeStruct((rows, DIM), jnp.float32),
            jax.ShapeDtypeStruct((8, 128), jnp.float32),
        ],
        compiler_params=pltpu.CompilerParams(
            dimension_semantics=("arbitrary",),
        ),
    )(xt, weight)

    c_loss = (1.0 + COMMIT) * loss[0, 0] / (B * C * H * W)
    return c_loss, q.reshape(B, H, W, C).transpose(0, 3, 1, 2)


# speedup vs baseline: 2.1466x; 1.0157x over previous
import jax
import jax.numpy as jnp
from jax.experimental import pallas as pl
from jax.experimental.pallas import tpu as pltpu

NUM_CODES = 1024
DIM = 64
COMMIT = 0.25
ROW_BLOCK = 4096


def _vq_kernel(xt_ref, w_ref, q_ref, loss_ref):
    b = pl.program_id(0)
    xt = xt_ref[...]
    w = w_ref[...]

    wsq = jnp.sum(w * w, axis=1)[None, :]
    xsq = jnp.sum(xt * xt, axis=1, keepdims=True)
    xw = jax.lax.dot_general(
        xt, w, (((1,), (1,)), ((), ())),
        preferred_element_type=jnp.float32)
    d2 = (xsq - 2.0 * xw) + wsq
    scores = jnp.sqrt(jnp.maximum(d2, 0.0))

    m = jnp.min(scores, axis=1, keepdims=True)
    iota = jax.lax.broadcasted_iota(jnp.int32, scores.shape, 1)
    masked = jnp.where(scores <= m, iota, NUM_CODES)
    idx = jnp.min(masked, axis=1, keepdims=True)

    onehot = (iota == idx).astype(jnp.float32)
    q = jax.lax.dot_general(
        onehot, w, (((1,), (0,)), ((), ())),
        preferred_element_type=jnp.float32)
    q_ref[...] = q

    partial = jnp.full((8, 128), jnp.sum(m * m), jnp.float32)

    @pl.when(b == 0)
    def _init():
        loss_ref[...] = jnp.zeros((8, 128), jnp.float32)

    loss_ref[...] += partial


@jax.jit
def kernel(inputs, weight):
    B, C, H, W = inputs.shape
    rows = B * H * W
    xt = inputs.transpose(0, 2, 3, 1).reshape(rows, C)

    q, loss = pl.pallas_call(
        _vq_kernel,
        grid=(rows // ROW_BLOCK,),
        in_specs=[
            pl.BlockSpec((ROW_BLOCK, DIM), lambda b: (b, 0)),
            pl.BlockSpec((NUM_CODES, DIM), lambda b: (0, 0)),
        ],
        out_specs=[
            pl.BlockSpec((ROW_BLOCK, DIM), lambda b: (b, 0)),
            pl.BlockSpec((8, 128), lambda b: (0, 0)),
        ],
        out_shape=[
            jax.ShapeDtypeStruct((rows, DIM), jnp.float32),
            jax.ShapeDtypeStruct((8, 128), jnp.float32),
        ],
        compiler_params=pltpu.CompilerParams(
            dimension_semantics=("arbitrary",),
        ),
    )(xt, weight)

    c_loss = (1.0 + COMMIT) * loss[0, 0] / (B * C * H * W)
    return c_loss, q.reshape(B, H, W, C).transpose(0, 3, 1, 2)
